# baseline (speedup 1.0000x reference)
import jax
import jax.numpy as jnp
from jax.experimental import pallas as pl
from jax.experimental.pallas import tpu as pltpu

T = 2048
D = 1024
E = 8
F = 512


def _router_body(x_ref, gw_ref, comb_ref):
    logits = jax.lax.dot_general(
        x_ref[...], gw_ref[...], (((1,), (1,)), ((), ())),
        preferred_element_type=jnp.float32)
    m = jnp.max(logits, axis=1, keepdims=True)
    ex = jnp.exp(logits - m)
    probs = ex / jnp.sum(ex, axis=1, keepdims=True)
    lane = jax.lax.broadcasted_iota(jnp.int32, (T, E), 1)
    rank = jnp.zeros((T, E), jnp.int32)
    for j in range(E):
        pj = probs[:, j:j + 1]
        beats = (pj > probs) | ((pj == probs) & (j < lane))
        rank = rank + beats.astype(jnp.int32)
    comb_ref[...] = probs * (rank < 2).astype(jnp.float32)


def _expert_body(xbf_ref, comb_ref, w1_ref, w3_ref, w2_ref, out_ref):
    e = pl.program_id(0)
    xs = xbf_ref[...]
    h1 = jax.lax.dot_general(xs, w1_ref[0], (((1,), (0,)), ((), ())),
                             preferred_element_type=jnp.float32)
    h3 = jax.lax.dot_general(xs, w3_ref[0], (((1,), (0,)), ((), ())),
                             preferred_element_type=jnp.float32)
    h = (h1 * jax.nn.sigmoid(h1) * h3).astype(jnp.bfloat16)
    y = jax.lax.dot_general(h, w2_ref[0], (((1,), (0,)), ((), ())),
                            preferred_element_type=jnp.float32)
    lane = jax.lax.broadcasted_iota(jnp.int32, (T, E), 1)
    w_e = jnp.sum(jnp.where(lane == e, comb_ref[...], 0.0), axis=1,
                  keepdims=True)
    yw = y * w_e

    @pl.when(e == 0)
    def _init():
        out_ref[...] = yw

    @pl.when(e != 0)
    def _acc():
        out_ref[...] = out_ref[...] + yw


def _moe(hidden_states, gate_w, w1, w3, w2):
    x = hidden_states.astype(jnp.float32)
    xbf = x.astype(jnp.bfloat16)
    w1t = jnp.swapaxes(w1, 1, 2).astype(jnp.bfloat16)
    w3t = jnp.swapaxes(w3, 1, 2).astype(jnp.bfloat16)
    w2t = jnp.swapaxes(w2, 1, 2).astype(jnp.bfloat16)

    comb = pl.pallas_call(
        _router_body,
        out_shape=jax.ShapeDtypeStruct((T, E), jnp.float32),
    )(x, gate_w.astype(jnp.float32))

    out = pl.pallas_call(
        _expert_body,
        grid=(E,),
        in_specs=[
            pl.BlockSpec((T, D), lambda e: (0, 0)),
            pl.BlockSpec((T, E), lambda e: (0, 0)),
            pl.BlockSpec((1, D, F), lambda e: (e, 0, 0)),
            pl.BlockSpec((1, D, F), lambda e: (e, 0, 0)),
            pl.BlockSpec((1, F, D), lambda e: (e, 0, 0)),
        ],
        out_specs=pl.BlockSpec((T, D), lambda e: (0, 0)),
        out_shape=jax.ShapeDtypeStruct((T, D), jnp.float32),
        compiler_params=pltpu.CompilerParams(
            dimension_semantics=("arbitrary",)),
    )(xbf, comb, w1t, w3t, w2t)
    return out


def kernel(hidden_states, num_global_tokens, max_num_tokens_per_gpu,
           gate_w, w1, w3, w2):
    del num_global_tokens, max_num_tokens_per_gpu
    return _moe(hidden_states, gate_w, w1, w3, w2)


# speedup vs baseline: 1.5043x; 1.5043x over previous
import jax
import jax.numpy as jnp
from jax.experimental import pallas as pl
from jax.experimental.pallas import tpu as pltpu

T = 2048
D = 1024
E = 8
F = 512


def _router_body(x_ref, gw_ref, comb_ref):
    logits = jax.lax.dot_general(
        x_ref[...], gw_ref[...], (((1,), (1,)), ((), ())),
        preferred_element_type=jnp.float32)
    m = jnp.max(logits, axis=1, keepdims=True)
    ex = jnp.exp(logits - m)
    probs = ex / jnp.sum(ex, axis=1, keepdims=True)
    lane = jax.lax.broadcasted_iota(jnp.int32, (T, E), 1)
    rank = jnp.zeros((T, E), jnp.int32)
    for j in range(E):
        pj = probs[:, j:j + 1]
        beats = (pj > probs) | ((pj == probs) & (j < lane))
        rank = rank + beats.astype(jnp.int32)
    comb_ref[...] = probs * (rank < 2).astype(jnp.float32)


def _expert_body(x_ref, comb_ref, w1_ref, w3_ref, w2_ref, out_ref, xbf_ref):
    e = pl.program_id(0)

    @pl.when(e == 0)
    def _cast():
        xbf_ref[...] = x_ref[...].astype(jnp.bfloat16)

    xs = xbf_ref[...]
    w1 = w1_ref[0].astype(jnp.bfloat16)
    w3 = w3_ref[0].astype(jnp.bfloat16)
    w2 = w2_ref[0].astype(jnp.bfloat16)
    h1 = jax.lax.dot_general(xs, w1, (((1,), (1,)), ((), ())),
                             preferred_element_type=jnp.float32)
    h3 = jax.lax.dot_general(xs, w3, (((1,), (1,)), ((), ())),
                             preferred_element_type=jnp.float32)
    h = (h1 * jax.nn.sigmoid(h1) * h3).astype(jnp.bfloat16)
    y = jax.lax.dot_general(h, w2, (((1,), (1,)), ((), ())),
                            preferred_element_type=jnp.float32)
    lane = jax.lax.broadcasted_iota(jnp.int32, (T, E), 1)
    w_e = jnp.sum(jnp.where(lane == e, comb_ref[...], 0.0), axis=1,
                  keepdims=True)
    yw = y * w_e

    @pl.when(e == 0)
    def _init():
        out_ref[...] = yw

    @pl.when(e != 0)
    def _acc():
        out_ref[...] = out_ref[...] + yw


def _moe(hidden_states, gate_w, w1, w3, w2):
    x = hidden_states.astype(jnp.float32)

    comb = pl.pallas_call(
        _router_body,
        out_shape=jax.ShapeDtypeStruct((T, E), jnp.float32),
    )(x, gate_w.astype(jnp.float32))

    out = pl.pallas_call(
        _expert_body,
        grid=(E,),
        in_specs=[
            pl.BlockSpec((T, D), lambda e: (0, 0)),
            pl.BlockSpec((T, E), lambda e: (0, 0)),
            pl.BlockSpec((1, F, D), lambda e: (e, 0, 0)),
            pl.BlockSpec((1, F, D), lambda e: (e, 0, 0)),
            pl.BlockSpec((1, D, F), lambda e: (e, 0, 0)),
        ],
        out_specs=pl.BlockSpec((T, D), lambda e: (0, 0)),
        out_shape=jax.ShapeDtypeStruct((T, D), jnp.float32),
        scratch_shapes=[pltpu.VMEM((T, D), jnp.bfloat16)],
        compiler_params=pltpu.CompilerParams(
            dimension_semantics=("arbitrary",)),
    )(x, comb, w1, w3, w2)
    return out


def kernel(hidden_states, num_global_tokens, max_num_tokens_per_gpu,
           gate_w, w1, w3, w2):
    del num_global_tokens, max_num_tokens_per_gpu
    return _moe(hidden_states, gate_w, w1, w3, w2)


# speedup vs baseline: 1.5987x; 1.0628x over previous
import jax
import jax.numpy as jnp
from jax.experimental import pallas as pl
from jax.experimental.pallas import tpu as pltpu

T = 2048
D = 1024
E = 8
F = 512


def _moe_body(x_ref, gw_ref, w1_ref, w3_ref, w2_ref, out_ref,
              comb_ref, xbf_ref):
    e = pl.program_id(0)

    @pl.when(e == 0)
    def _router():
        xbf_ref[...] = x_ref[...].astype(jnp.bfloat16)
        logits = jax.lax.dot_general(
            x_ref[...], gw_ref[...], (((1,), (1,)), ((), ())),
            preferred_element_type=jnp.float32)
        m = jnp.max(logits, axis=1, keepdims=True)
        ex = jnp.exp(logits - m)
        probs = ex / jnp.sum(ex, axis=1, keepdims=True)
        lane = jax.lax.broadcasted_iota(jnp.int32, (T, E), 1)
        rank = jnp.zeros((T, E), jnp.int32)
        for j in range(E):
            pj = probs[:, j:j + 1]
            beats = (pj > probs) | ((pj == probs) & (j < lane))
            rank = rank + beats.astype(jnp.int32)
        comb_ref[...] = probs * (rank < 2).astype(jnp.float32)

    xs = xbf_ref[...]
    w1 = w1_ref[0].astype(jnp.bfloat16)
    w3 = w3_ref[0].astype(jnp.bfloat16)
    w2 = w2_ref[0].astype(jnp.bfloat16)
    h1 = jax.lax.dot_general(xs, w1, (((1,), (1,)), ((), ())),
                             preferred_element_type=jnp.float32)
    h3 = jax.lax.dot_general(xs, w3, (((1,), (1,)), ((), ())),
                             preferred_element_type=jnp.float32)
    h = (h1 * jax.nn.sigmoid(h1) * h3).astype(jnp.bfloat16)
    y = jax.lax.dot_general(h, w2, (((1,), (1,)), ((), ())),
                            preferred_element_type=jnp.float32)
    lane = jax.lax.broadcasted_iota(jnp.int32, (T, E), 1)
    w_e = jnp.sum(jnp.where(lane == e, comb_ref[...], 0.0), axis=1,
                  keepdims=True)
    yw = y * w_e

    @pl.when(e == 0)
    def _init():
        out_ref[...] = yw

    @pl.when(e != 0)
    def _acc():
        out_ref[...] = out_ref[...] + yw


def _moe(hidden_states, gate_w, w1, w3, w2):
    x = hidden_states.astype(jnp.float32)
    out = pl.pallas_call(
        _moe_body,
        grid=(E,),
        in_specs=[
            pl.BlockSpec((T, D), lambda e: (0, 0)),
            pl.BlockSpec((E, D), lambda e: (0, 0)),
            pl.BlockSpec((1, F, D), lambda e: (e, 0, 0)),
            pl.BlockSpec((1, F, D), lambda e: (e, 0, 0)),
            pl.BlockSpec((1, D, F), lambda e: (e, 0, 0)),
        ],
        out_specs=pl.BlockSpec((T, D), lambda e: (0, 0)),
        out_shape=jax.ShapeDtypeStruct((T, D), jnp.float32),
        scratch_shapes=[pltpu.VMEM((T, E), jnp.float32),
                        pltpu.VMEM((T, D), jnp.bfloat16)],
        compiler_params=pltpu.CompilerParams(
            dimension_semantics=("arbitrary",)),
    )(x, gate_w.astype(jnp.float32), w1, w3, w2)
    return out


def kernel(hidden_states, num_global_tokens, max_num_tokens_per_gpu,
           gate_w, w1, w3, w2):
    del num_global_tokens, max_num_tokens_per_gpu
    return _moe(hidden_states, gate_w, w1, w3, w2)
